# Initial kernel scaffold; baseline (speedup 1.0000x reference)
import jax
import jax.numpy as jnp
from jax.experimental import pallas as pl


def kernel(x, weight, bias):
    raise NotImplementedError("write your pallas kernel here")


# speedup vs baseline: 1.4243x; 1.4243x over previous
import functools

import jax
import jax.numpy as jnp
from jax import lax
from jax.experimental import pallas as pl
from jax.experimental.pallas import tpu as pltpu
from jax.experimental.pallas import tpu_sc as plsc

B = 16384
F = 26
FIELD_DIM = 38461
NC, NS, L = 2, 16, 16
NW = NC * NS
RPW = B // NW
IPW = RPW * F
CHUNK = 128
NCHUNK = IPW // CHUNK
CPF = RPW // CHUNK


def _body(xtw_hbm, wflat_hbm, bias_hbm, out_hbm, xbuf, gbuf, bias_v, obuf, sem):
    wid = lax.axis_index("s") * NC + lax.axis_index("c")

    pltpu.sync_copy(xtw_hbm.at[wid], xbuf)
    pltpu.sync_copy(bias_hbm, bias_v)

    def fire(k, _):
        f = k // CPF
        off = (f * FIELD_DIM).astype(jnp.int32)
        for c in range(CHUNK // L):
            sl = pl.ds(c * L, L)
            xbuf[k, sl] = xbuf[k, sl] + off
        pltpu.async_copy(
            wflat_hbm.at[xbuf.at[k]], gbuf.at[pl.ds(k * CHUNK, CHUNK)], sem
        )
        return 0

    lax.fori_loop(0, NCHUNK, fire, 0)

    pltpu.make_async_copy(wflat_hbm.at[pl.ds(0, IPW)], gbuf, sem).wait()

    bias_vec = bias_v[...]

    def reduce(j, _):
        base = j * L
        vacc = bias_vec
        for f in range(F):
            vacc = vacc + gbuf[pl.ds(f * RPW + base, L)]
        obuf[pl.ds(base, L)] = 1.0 / (1.0 + jnp.exp(-vacc))
        return 0

    lax.fori_loop(0, RPW // L, reduce, 0)

    pltpu.sync_copy(obuf, out_hbm.at[pl.ds(wid * RPW, RPW)])


@jax.jit
def kernel(x, weight, bias):
    xtw = (
        x.reshape(NW, RPW, F)
        .swapaxes(1, 2)
        .reshape(NW, NCHUNK, CHUNK)
    )
    wflat = weight.reshape(-1)
    bias16 = jnp.broadcast_to(bias, (L,))

    mesh = plsc.VectorSubcoreMesh(core_axis_name="c", subcore_axis_name="s")
    run = pl.kernel(
        _body,
        out_type=jax.ShapeDtypeStruct((B,), jnp.float32),
        mesh=mesh,
        scratch_types=[
            pltpu.VMEM((NCHUNK, CHUNK), jnp.int32),
            pltpu.VMEM((IPW,), jnp.float32),
            pltpu.VMEM((L,), jnp.float32),
            pltpu.VMEM((RPW,), jnp.float32),
            pltpu.SemaphoreType.DMA,
        ],
    )
    return run(xtw, wflat, bias16)
